# Initial kernel scaffold; baseline (speedup 1.0000x reference)
import jax
import jax.numpy as jnp
from jax.experimental import pallas as pl


def kernel(x, edge_index, W, b):
    raise NotImplementedError("write your pallas kernel here")


# speedup vs baseline: 3.5946x; 3.5946x over previous
import functools

import jax
import jax.numpy as jnp
from jax import lax
from jax.experimental import pallas as pl
from jax.experimental.pallas import tpu as pltpu
from jax.experimental.pallas import tpu_sc as plsc

_NC = 2
_NS = 16
_NW = _NC * _NS


def _tc_matmul(x, W, b_half):
    n, d_in = x.shape
    d_out = W.shape[1]

    def body(x_ref, w_ref, b_ref, o_ref):
        o_ref[...] = (
            jnp.dot(x_ref[...], w_ref[...], preferred_element_type=jnp.float32)
            + b_ref[...]
        )

    return pl.pallas_call(
        body,
        out_shape=jax.ShapeDtypeStruct((n, d_out), jnp.float32),
    )(x, W, b_half)


def _make_sc_gather_add(E, D, C):
    epw = E // _NW
    nchunks = epw // C
    mesh = plsc.VectorSubcoreMesh(core_axis_name="c", subcore_axis_name="s")

    @functools.partial(
        pl.kernel,
        mesh=mesh,
        out_type=jax.ShapeDtypeStruct((E, D), jnp.float32),
        scratch_types=[
            pltpu.VMEM((C,), jnp.int32),
            pltpu.VMEM((C,), jnp.int32),
            pltpu.VMEM((C, D), jnp.float32),
            pltpu.VMEM((C, D), jnp.float32),
            pltpu.SemaphoreType.DMA,
            pltpu.SemaphoreType.DMA,
        ],
    )
    def sc_fn(y_hbm, row_hbm, col_hbm, out_hbm, idxr, idxc, bufa, bufb, sema, semb):
        wid = lax.axis_index("s") * _NC + lax.axis_index("c")
        base = wid * epw

        def chunk_body(j, carry):
            off = base + j * C
            pltpu.sync_copy(row_hbm.at[pl.ds(off, C)], idxr)
            pltpu.sync_copy(col_hbm.at[pl.ds(off, C)], idxc)
            ca = pltpu.async_copy(y_hbm.at[idxr], bufa, sema)
            cb = pltpu.async_copy(y_hbm.at[idxc], bufb, semb)
            ca.wait()
            cb.wait()

            def add_body(e, c2):
                for k in range(D // 16):
                    sl = pl.ds(k * 16, 16)
                    plsc.addupdate(bufa.at[e, sl], bufb[e, sl])
                return c2

            lax.fori_loop(0, C, add_body, 0, unroll=False)
            pltpu.sync_copy(bufa, out_hbm.at[pl.ds(off, C)])
            return carry

        lax.fori_loop(0, nchunks, chunk_body, 0, unroll=False)

    return sc_fn


def kernel(x, edge_index, W, b):
    n, d_in = x.shape
    d_out = W.shape[1]
    E = edge_index.shape[1]

    b_half = (0.5 * b).reshape(1, d_out).astype(jnp.float32)
    y = _tc_matmul(x, W, b_half)

    C = 80
    sc_fn = _make_sc_gather_add(E, d_out, C)
    row = edge_index[0]
    col = edge_index[1]
    return sc_fn(y, row, col)


# speedup vs baseline: 7.6297x; 2.1225x over previous
import functools

import jax
import jax.numpy as jnp
from jax import lax
from jax.experimental import pallas as pl
from jax.experimental.pallas import tpu as pltpu
from jax.experimental.pallas import tpu_sc as plsc

_NC = 2
_NS = 16
_NW = _NC * _NS


def _tc_matmul(x, W, b_half):
    n, d_in = x.shape
    d_out = W.shape[1]

    def body(x_ref, w_ref, b_ref, o_ref):
        o_ref[...] = (
            jnp.dot(x_ref[...], w_ref[...], preferred_element_type=jnp.float32)
            + b_ref[...]
        )

    return pl.pallas_call(
        body,
        out_shape=jax.ShapeDtypeStruct((n, d_out), jnp.float32),
    )(x, W, b_half)


def _make_sc_gather_add(E, D, C, NBUF):
    epw = E // _NW
    nchunks = epw // C
    nouter = nchunks // NBUF
    main = nouter * NBUF
    ntail = nchunks - main
    assert ntail == 2 and NBUF >= 4 and epw % C == 0 and C % 8 == 0 and C <= 128
    mesh = plsc.VectorSubcoreMesh(core_axis_name="c", subcore_axis_name="s")

    @functools.partial(
        pl.kernel,
        mesh=mesh,
        out_type=jax.ShapeDtypeStruct((E, D), jnp.float32),
        scratch_types=[
            pltpu.VMEM((epw,), jnp.int32),
            pltpu.VMEM((epw,), jnp.int32),
            pltpu.VMEM((NBUF, C, D), jnp.float32),
            pltpu.VMEM((NBUF, C, D), jnp.float32),
            pltpu.SemaphoreType.DMA((NBUF,)),
            pltpu.SemaphoreType.DMA((NBUF,)),
        ],
    )
    def sc_fn(y_hbm, row_hbm, col_hbm, out_hbm, idxr, idxc, bufa, bufb, gsem, wsem):
        wid = lax.axis_index("s") * _NC + lax.axis_index("c")
        base = wid * epw

        pltpu.sync_copy(row_hbm.at[pl.ds(base, epw)], idxr)
        pltpu.sync_copy(col_hbm.at[pl.ds(base, epw)], idxc)

        def fire_gather(j, s):
            o = j * C
            pltpu.async_copy(y_hbm.at[idxr.at[pl.ds(o, C)]], bufa.at[s], gsem.at[s])
            pltpu.async_copy(y_hbm.at[idxc.at[pl.ds(o, C)]], bufb.at[s], gsem.at[s])

        def wait_gather(j, s):
            o = j * C
            pltpu.make_async_copy(
                y_hbm.at[idxr.at[pl.ds(o, C)]], bufa.at[s], gsem.at[s]).wait()
            pltpu.make_async_copy(
                y_hbm.at[idxc.at[pl.ds(o, C)]], bufb.at[s], gsem.at[s]).wait()

        def fire_write(j, s):
            o = base + j * C
            pltpu.async_copy(bufa.at[s], out_hbm.at[pl.ds(o, C)], wsem.at[s])

        def wait_write(j, s):
            o = base + j * C
            pltpu.make_async_copy(
                bufa.at[s], out_hbm.at[pl.ds(o, C)], wsem.at[s]).wait()

        def do_add(s):
            def add_body(e, c2):
                for k in range(D // 16):
                    sl = pl.ds(k * 16, 16)
                    plsc.addupdate(bufa.at[s, e, sl], bufb[s, e, sl])
                return c2

            lax.fori_loop(0, C, add_body, 0, unroll=False)

        fire_gather(0, 0)
        fire_gather(1, 1)

        def outer(jj, carry):
            for s in range(NBUF):
                j = jj * NBUF + s
                if s < 2:
                    @pl.when(jj >= 1)
                    def _w():
                        wait_write(j - 2, (s + 2) % NBUF)
                else:
                    wait_write(j - 2, s - 2)
                fire_gather(j + 2, (s + 2) % NBUF)
                wait_gather(j, s)
                do_add(s)
                fire_write(j, s)
            return carry

        lax.fori_loop(0, nouter, outer, 0, unroll=False)

        for t in range(ntail):
            j = main + t
            wait_write(j - 2, (t + 2) % NBUF)
            wait_gather(j, t)
            do_add(t)
            fire_write(j, t)
        for t in range(ntail):
            wait_write(main + t, t)

    return sc_fn


def kernel(x, edge_index, W, b):
    n, d_in = x.shape
    d_out = W.shape[1]
    E = edge_index.shape[1]

    b_half = (0.5 * b).reshape(1, d_out).astype(jnp.float32)
    y = _tc_matmul(x, W, b_half)

    sc_fn = _make_sc_gather_add(E, d_out, C=40, NBUF=4)
    row = edge_index[0]
    col = edge_index[1]
    return sc_fn(y, row, col)
